# baseline (speedup 1.0000x reference)
import jax
import jax.numpy as jnp
from jax.experimental import pallas as pl

N = 10000
G = 64


def _proj_body(hg_ref, wp_ref, bp_ref, out_ref):
    out_ref[...] = hg_ref[...] @ wp_ref[...] + bp_ref[...]


def _bn(h, g, b):
    mu = jnp.mean(h, axis=0)
    var = jnp.var(h, axis=0)
    return (h - mu) / jnp.sqrt(var + 1e-5) * g + b


def _genconv(h, src, dst, edge_emb, W, b, t):
    m = jax.nn.relu(h[src] + edge_emb) + 1e-7
    logits = m * t
    seg_max = jax.ops.segment_max(logits, dst, num_segments=N)
    w = jnp.exp(logits - seg_max[dst])
    den = jax.ops.segment_sum(w, dst, num_segments=N)
    alpha = w / (den[dst] + 1e-16)
    aggr = jax.ops.segment_sum(alpha * m, dst, num_segments=N)
    return (h + aggr) @ W + b


def kernel(x, edge_index, edge_attr, batch, W0, b0, We, be, gcn_W, gcn_b, t, gamma, beta, Wp, bp):
    L = gcn_W.shape[0]
    src = edge_index[0]
    dst = edge_index[1]
    h = x @ W0 + b0
    edge_emb = edge_attr @ We + be
    h = _genconv(h, src, dst, edge_emb, gcn_W[0], gcn_b[0], t[0])
    for l in range(1, L):
        h1 = _bn(h, gamma[l - 1], beta[l - 1])
        h2 = jax.nn.relu(h1)
        h = _genconv(h2, src, dst, edge_emb, gcn_W[l], gcn_b[l], t[l]) + h
    h = _bn(h, gamma[L - 1], beta[L - 1])
    counts = jax.ops.segment_sum(jnp.ones((N,), dtype=h.dtype), batch, num_segments=G)
    hsum = jax.ops.segment_sum(h, batch, num_segments=G)
    h_graph = hsum / jnp.maximum(counts, 1.0)[:, None]
    return pl.pallas_call(
        _proj_body,
        out_shape=jax.ShapeDtypeStruct((G, Wp.shape[1]), jnp.float32),
    )(h_graph, Wp, bp)


# speedup vs baseline: 1.4372x; 1.4372x over previous
import functools

import jax
import jax.numpy as jnp
from jax import lax
from jax.experimental import pallas as pl
from jax.experimental.pallas import tpu as pltpu
from jax.experimental.pallas import tpu_sc as plsc

N = 10000
E = 320000
H = 256
G = 64
NWORK = 32
NGRP = 625
EB = 128
NEG = -1e30

_GSTART = [(w * NGRP) // NWORK for w in range(NWORK + 1)]


def _sc_agg_body(g_hbm, ee_hbm, src_hbm, dst_hbm, eoff_hbm, tvec_hbm, out_hbm,
                 eoff_v, tvec_v, src_v, dst_v, rows_v, ee_v,
                 rmax_v, den_v, num_v, out_v, sem):
    c = lax.axis_index("c")
    s = lax.axis_index("s")
    wid = s * 2 + c

    pltpu.sync_copy(eoff_hbm, eoff_v)
    pltpu.sync_copy(tvec_hbm, tvec_v)
    tv = tvec_v[...]

    ev = eoff_v[pl.ds(2 * wid, 16)]
    e0 = ev[0]
    g0 = ev[1]
    e1 = ev[2]
    g1 = ev[3]

    zv = jnp.zeros((16,), jnp.float32)

    def _zero_out_v(i, _):
        for cg in range(16):
            out_v[i, pl.ds(cg * 16, 16)] = zv
        return 0

    lax.fori_loop(0, 16, _zero_out_v, 0)

    def _zfill(k, _):
        pltpu.sync_copy(out_v, out_hbm.at[pl.ds((g0 + k) * 16, 16)])
        return 0

    lax.fori_loop(0, g1 - g0, _zfill, 0)

    def _reset_acc():
        neg = jnp.full((16,), NEG, jnp.float32)
        for g in range(16):
            rmax_v[g] = neg
            den_v[g] = zv
            num_v[g] = zv

    def _finalize(cur, curg):
        r = cur - curg * 16
        for g in range(16):
            out_v[r, pl.ds(g * 16, 16)] = num_v[g] / den_v[g]

    _reset_acc()

    e_al = (e0 // EB) * EB
    nblk = (e1 - e_al + EB - 1) // EB

    def _block(k, car):
        cur, curg = car
        base = e_al + k * EB
        pltpu.sync_copy(src_hbm.at[pl.ds(base, EB)], src_v)
        pltpu.sync_copy(dst_hbm.at[pl.ds(base, EB)], dst_v.at[pl.ds(0, EB)])
        pltpu.async_copy(g_hbm.at[src_v], rows_v, sem).wait()
        pltpu.sync_copy(ee_hbm.at[pl.ds(base, EB)], ee_v)
        j_lo = jnp.maximum(e0 - base, 0)
        j_hi = jnp.minimum(e1 - base, EB)

        def _edge(j, car2):
            cur2, curg2 = car2
            d = dst_v[pl.ds(j, 16)][0]

            def _switch(cc, gg):
                ng = d // 16

                @pl.when(cc >= 0)
                def _():
                    _finalize(cc, gg)

                @pl.when(jnp.logical_and(cc >= 0, ng != gg))
                def _():
                    pltpu.sync_copy(out_v, out_hbm.at[pl.ds(gg * 16, 16)])
                    lax.fori_loop(0, 16, _zero_out_v, 0)

                _reset_acc()
                return d, ng

            def _keep(cc, gg):
                return cc, gg

            cur2, curg2 = lax.cond(d != cur2, _switch, _keep, cur2, curg2)

            for g in range(16):
                hrow = rows_v[j, pl.ds(g * 16, 16)]
                erow = ee_v[j, pl.ds(g * 16, 16)]
                m = jnp.maximum(hrow + erow, 0.0) + 1e-7
                logit = m * tv
                rm = rmax_v[g]
                nm = jnp.maximum(rm, logit)
                a = jnp.exp(logit - nm)
                sc = jnp.exp(rm - nm)
                den_v[g] = den_v[g] * sc + a
                num_v[g] = num_v[g] * sc + a * m
                rmax_v[g] = nm
            return cur2, curg2

        return lax.fori_loop(j_lo, j_hi, _edge, (cur, curg))

    cur, curg = lax.fori_loop(0, nblk, _block, (jnp.int32(-1), g0))

    @pl.when(cur >= 0)
    def _():
        _finalize(cur, curg)
        pltpu.sync_copy(out_v, out_hbm.at[pl.ds(curg * 16, 16)])


@functools.partial(jax.jit, static_argnames=())
def _sc_agg(g, ee, src_s, dst_s, eoff, tvec):
    mesh = plsc.VectorSubcoreMesh(core_axis_name="c", subcore_axis_name="s",
                                  num_cores=2, num_subcores=16)
    return pl.kernel(
        _sc_agg_body,
        out_type=jax.ShapeDtypeStruct((N, H), jnp.float32),
        mesh=mesh,
        scratch_types=[
            pltpu.VMEM((80,), jnp.int32),
            pltpu.VMEM((16,), jnp.float32),
            pltpu.VMEM((EB,), jnp.int32),
            pltpu.VMEM((EB + 16,), jnp.int32),
            pltpu.VMEM((EB, H), jnp.float32),
            pltpu.VMEM((EB, H), jnp.float32),
            pltpu.VMEM((16, 16), jnp.float32),
            pltpu.VMEM((16, 16), jnp.float32),
            pltpu.VMEM((16, 16), jnp.float32),
            pltpu.VMEM((16, H), jnp.float32),
            pltpu.SemaphoreType.DMA,
        ],
    )(g, ee, src_s, dst_s, eoff, tvec)


def _mm_body(x_ref, w_ref, b_ref, o_ref):
    o_ref[...] = (jnp.dot(x_ref[...], w_ref[...],
                          preferred_element_type=jnp.float32) + b_ref[...])


def _mm(x, w, b, blk):
    m, k = x.shape
    n = w.shape[1]
    return pl.pallas_call(
        _mm_body,
        grid=(m // blk,),
        in_specs=[pl.BlockSpec((blk, k), lambda i: (i, 0)),
                  pl.BlockSpec((k, n), lambda i: (0, 0)),
                  pl.BlockSpec((1, n), lambda i: (0, 0))],
        out_specs=pl.BlockSpec((blk, n), lambda i: (i, 0)),
        out_shape=jax.ShapeDtypeStruct((m, n), jnp.float32),
    )(x, w, b.reshape(1, -1))


def _layer_body(g_ref, a_ref, w_ref, b_ref, r_ref, h_ref, s_ref):
    h = (jnp.dot(g_ref[...] + a_ref[...], w_ref[...],
                 preferred_element_type=jnp.float32) + b_ref[...] + r_ref[...])
    h_ref[...] = h
    cs = jnp.sum(h, axis=0, keepdims=True)
    cq = jnp.sum(h * h, axis=0, keepdims=True)
    st = jnp.concatenate([cs, cq, jnp.zeros((6, h.shape[1]), jnp.float32)], 0)

    @pl.when(pl.program_id(0) == 0)
    def _():
        s_ref[...] = st

    @pl.when(pl.program_id(0) > 0)
    def _():
        s_ref[...] = s_ref[...] + st


def _layer(g, aggr, w, b, res, blk=1000):
    return pl.pallas_call(
        _layer_body,
        grid=(N // blk,),
        in_specs=[pl.BlockSpec((blk, H), lambda i: (i, 0)),
                  pl.BlockSpec((blk, H), lambda i: (i, 0)),
                  pl.BlockSpec((H, H), lambda i: (0, 0)),
                  pl.BlockSpec((1, H), lambda i: (0, 0)),
                  pl.BlockSpec((blk, H), lambda i: (i, 0))],
        out_specs=[pl.BlockSpec((blk, H), lambda i: (i, 0)),
                   pl.BlockSpec((8, H), lambda i: (0, 0))],
        out_shape=[jax.ShapeDtypeStruct((N, H), jnp.float32),
                   jax.ShapeDtypeStruct((8, H), jnp.float32)],
    )(g, aggr, w, b.reshape(1, -1), res)


def _bnrelu_body(relu, h_ref, s_ref, gm_ref, bt_ref, o_ref):
    mu = s_ref[0:1, :] / N
    var = s_ref[1:2, :] / N - mu * mu
    rstd = lax.rsqrt(var + 1e-5)
    o = (h_ref[...] - mu) * rstd * gm_ref[...] + bt_ref[...]
    if relu:
        o = jnp.maximum(o, 0.0)
    o_ref[...] = o


def _bnrelu(h, stats, gm, bt, relu, blk=1000):
    return pl.pallas_call(
        functools.partial(_bnrelu_body, relu),
        grid=(N // blk,),
        in_specs=[pl.BlockSpec((blk, H), lambda i: (i, 0)),
                  pl.BlockSpec((8, H), lambda i: (0, 0)),
                  pl.BlockSpec((1, H), lambda i: (0, 0)),
                  pl.BlockSpec((1, H), lambda i: (0, 0))],
        out_specs=pl.BlockSpec((blk, H), lambda i: (i, 0)),
        out_shape=jax.ShapeDtypeStruct((N, H), jnp.float32),
    )(h, stats, gm.reshape(1, -1), bt.reshape(1, -1))


def _pool_body(msel_ref, h_ref, wp_ref, bp_ref, o_ref, acc, cnt):
    @pl.when(pl.program_id(0) == 0)
    def _():
        acc[...] = jnp.zeros_like(acc)
        cnt[...] = jnp.zeros_like(cnt)

    msel_t = msel_ref[...]
    acc[...] = acc[...] + lax.dot_general(
        msel_t, h_ref[...], (((0,), (0,)), ((), ())),
        preferred_element_type=jnp.float32)
    cnt[...] = cnt[...] + jnp.broadcast_to(
        jnp.sum(msel_t, axis=0)[:, None], cnt.shape)

    @pl.when(pl.program_id(0) == pl.num_programs(0) - 1)
    def _():
        hg = acc[...] / jnp.maximum(cnt[...][:, 0:1], 1.0)
        o_ref[...] = (jnp.dot(hg, wp_ref[...],
                              preferred_element_type=jnp.float32) + bp_ref[...])


def _pool_proj(msel, hn, wp, bp, blk=2000):
    t_out = wp.shape[1]
    return pl.pallas_call(
        _pool_body,
        grid=(N // blk,),
        in_specs=[pl.BlockSpec((blk, G), lambda i: (i, 0)),
                  pl.BlockSpec((blk, H), lambda i: (i, 0)),
                  pl.BlockSpec((H, t_out), lambda i: (0, 0)),
                  pl.BlockSpec((1, t_out), lambda i: (0, 0))],
        out_specs=pl.BlockSpec((G, t_out), lambda i: (0, 0)),
        out_shape=jax.ShapeDtypeStruct((G, t_out), jnp.float32),
        scratch_shapes=[pltpu.VMEM((G, H), jnp.float32),
                        pltpu.VMEM((G, 128), jnp.float32)],
    )(msel, hn, wp, bp.reshape(1, -1))


def kernel(x, edge_index, edge_attr, batch, W0, b0, We, be, gcn_W, gcn_b, t,
           gamma, beta, Wp, bp):
    L = gcn_W.shape[0]
    src = edge_index[0]
    dst = edge_index[1]

    perm = jnp.argsort(dst)
    dst_s = dst[perm].astype(jnp.int32)
    src_s = src[perm].astype(jnp.int32)
    ea_s = edge_attr[perm]
    row_off = jnp.searchsorted(dst_s, jnp.arange(N + 1, dtype=jnp.int32)
                               ).astype(jnp.int32)
    starts = jnp.array([16 * g for g in _GSTART], dtype=jnp.int32)
    eo = row_off[starts]
    gidx = jnp.array(_GSTART, dtype=jnp.int32)
    eoff = jnp.zeros((80,), jnp.int32)
    eoff = eoff.at[0:66:2].set(eo).at[1:66:2].set(gidx)

    h0 = _mm(x, W0, b0, blk=1000)
    ee = _mm(ea_s, We, be, blk=2000)

    zero_res = jnp.zeros((N, H), jnp.float32)
    g = h0
    res = zero_res
    h = None
    stats = None
    for l in range(L):
        tvec = jnp.full((16,), t[l], jnp.float32)
        aggr = _sc_agg(g, ee, src_s, dst_s, eoff, tvec)
        h, stats = _layer(g, aggr, gcn_W[l], gcn_b[l], res)
        res = h
        if l < L - 1:
            g = _bnrelu(h, stats, gamma[l], beta[l], relu=True)
    hn = _bnrelu(h, stats, gamma[L - 1], beta[L - 1], relu=False)

    msel_t = (batch[:, None] == jnp.arange(G, dtype=batch.dtype)[None, :]
              ).astype(jnp.float32)
    return _pool_proj(msel_t, hn, Wp, bp)


# speedup vs baseline: 1.4841x; 1.0326x over previous
import functools

import jax
import jax.numpy as jnp
from jax import lax
from jax.experimental import pallas as pl
from jax.experimental.pallas import tpu as pltpu
from jax.experimental.pallas import tpu_sc as plsc

N = 10000
E = 320000
H = 256
G = 64
NWORK = 32
NGRP = 625
EB = 128
RB = 336
NEG = -1e30

_GSTART = [(w * NGRP) // NWORK for w in range(NWORK + 1)]


def _sc_agg_body(g_hbm, ee_hbm, src_hbm, dst_hbm, roff_hbm, eoff_hbm,
                 tvec_hbm, out_hbm,
                 eoff_v, tvec_v, roff_v, src_v, dst_v, rows_v, ee_v,
                 rmax_v, den_v, num_v, out_v, sem):
    c = lax.axis_index("c")
    s = lax.axis_index("s")
    wid = s * 2 + c

    pltpu.sync_copy(eoff_hbm, eoff_v)
    pltpu.sync_copy(tvec_hbm, tvec_v)
    tv = tvec_v[...]

    ev = eoff_v[pl.ds(2 * wid, 16)]
    e0 = ev[0]
    g0 = ev[1]
    e1 = ev[2]
    g1 = ev[3]
    n0 = g0 * 16

    pltpu.sync_copy(roff_hbm.at[pl.ds(n0, RB)], roff_v)

    zv = jnp.zeros((16,), jnp.float32)
    negv = jnp.full((16,), NEG, jnp.float32)

    def _zero_out(i, _):
        for cg in range(16):
            out_v[i, pl.ds(cg * 16, 16)] = zv
        return 0

    lax.fori_loop(0, 16, _zero_out, 0)

    def _zfill(k, _):
        pltpu.sync_copy(out_v, out_hbm.at[pl.ds((g0 + k) * 16, 16)])
        return 0

    lax.fori_loop(0, g1 - g0, _zfill, 0)

    def _reset_acc():
        for g in range(16):
            rmax_v[g] = negv
            den_v[g] = zv
            num_v[g] = zv

    e_al = (e0 // EB) * EB
    nblk = (e1 - e_al + EB - 1) // EB

    def _block(k, car):
        cur, curg = car
        base = e_al + k * EB
        pltpu.sync_copy(src_hbm.at[pl.ds(base, EB)], src_v)
        pltpu.sync_copy(dst_hbm.at[pl.ds(base, EB)], dst_v.at[pl.ds(0, EB)])
        pltpu.async_copy(g_hbm.at[src_v], rows_v, sem).wait()
        pltpu.sync_copy(ee_hbm.at[pl.ds(base, EB)], ee_v)
        j_lo = jnp.maximum(e0 - base, 0)
        j_hi = jnp.minimum(e1 - base, EB)
        last = dst_v[pl.ds(j_hi - 1, 16)][0]

        def _node(nd, cg):
            rv = roff_v[pl.ds(nd - n0, 16)]
            es = rv[0]
            ee2 = rv[1]
            js = jnp.maximum(es - base, j_lo)
            je = jnp.minimum(ee2 - base, j_hi)
            ng = nd // 16

            @pl.when(ng != cg)
            def _():
                pltpu.sync_copy(out_v, out_hbm.at[pl.ds(cg * 16, 16)])
                lax.fori_loop(0, 16, _zero_out, 0)

            @pl.when(es >= base + j_lo)
            def _():
                _reset_acc()

            def _edge(j, _):
                for g in range(16):
                    hrow = rows_v[j, pl.ds(g * 16, 16)]
                    erow = ee_v[j, pl.ds(g * 16, 16)]
                    m = jnp.maximum(hrow + erow, 0.0) + 1e-7
                    logit = m * tv
                    rm = rmax_v[g]
                    nm = jnp.maximum(rm, logit)
                    a = jnp.exp(logit - nm)
                    sc = jnp.exp(rm - nm)
                    den_v[g] = den_v[g] * sc + a
                    num_v[g] = num_v[g] * sc + a * m
                    rmax_v[g] = nm
                return 0

            lax.fori_loop(js, je, _edge, 0)

            @pl.when(jnp.logical_and(ee2 <= base + j_hi, ee2 > es))
            def _():
                r = nd - ng * 16
                for g in range(16):
                    out_v[r, pl.ds(g * 16, 16)] = num_v[g] / den_v[g]

            return ng

        curg = lax.fori_loop(cur, last + 1, _node, curg)
        rl = roff_v[pl.ds(last - n0, 16)]
        cur = lax.select(rl[1] <= base + EB, last + 1, last)
        return cur, curg

    cur, curg = lax.fori_loop(0, nblk, _block, (n0, g0))
    pltpu.sync_copy(out_v, out_hbm.at[pl.ds(curg * 16, 16)])


def _sc_agg(g, ee, src_s, dst_s, roff_pad, eoff, tvec):
    mesh = plsc.VectorSubcoreMesh(core_axis_name="c", subcore_axis_name="s",
                                  num_cores=2, num_subcores=16)
    return pl.kernel(
        _sc_agg_body,
        out_type=jax.ShapeDtypeStruct((N, H), jnp.float32),
        mesh=mesh,
        scratch_types=[
            pltpu.VMEM((80,), jnp.int32),
            pltpu.VMEM((16,), jnp.float32),
            pltpu.VMEM((RB,), jnp.int32),
            pltpu.VMEM((EB,), jnp.int32),
            pltpu.VMEM((EB + 16,), jnp.int32),
            pltpu.VMEM((EB, H), jnp.float32),
            pltpu.VMEM((EB, H), jnp.float32),
            pltpu.VMEM((16, 16), jnp.float32),
            pltpu.VMEM((16, 16), jnp.float32),
            pltpu.VMEM((16, 16), jnp.float32),
            pltpu.VMEM((16, H), jnp.float32),
            pltpu.SemaphoreType.DMA,
        ],
    )(g, ee, src_s, dst_s, roff_pad, eoff, tvec)


def _mm_body(x_ref, w_ref, b_ref, o_ref):
    o_ref[...] = (jnp.dot(x_ref[...], w_ref[...],
                          preferred_element_type=jnp.float32) + b_ref[...])


def _mm(x, w, b, blk):
    m, k = x.shape
    n = w.shape[1]
    return pl.pallas_call(
        _mm_body,
        grid=(m // blk,),
        in_specs=[pl.BlockSpec((blk, k), lambda i: (i, 0)),
                  pl.BlockSpec((k, n), lambda i: (0, 0)),
                  pl.BlockSpec((1, n), lambda i: (0, 0))],
        out_specs=pl.BlockSpec((blk, n), lambda i: (i, 0)),
        out_shape=jax.ShapeDtypeStruct((m, n), jnp.float32),
    )(x, w, b.reshape(1, -1))


def _layer_body(g_ref, a_ref, w_ref, b_ref, r_ref, h_ref, s_ref):
    h = (jnp.dot(g_ref[...] + a_ref[...], w_ref[...],
                 preferred_element_type=jnp.float32) + b_ref[...] + r_ref[...])
    h_ref[...] = h
    cs = jnp.sum(h, axis=0, keepdims=True)
    cq = jnp.sum(h * h, axis=0, keepdims=True)
    st = jnp.concatenate([cs, cq, jnp.zeros((6, h.shape[1]), jnp.float32)], 0)

    @pl.when(pl.program_id(0) == 0)
    def _():
        s_ref[...] = st

    @pl.when(pl.program_id(0) > 0)
    def _():
        s_ref[...] = s_ref[...] + st


def _layer(g, aggr, w, b, res, blk=1000):
    return pl.pallas_call(
        _layer_body,
        grid=(N // blk,),
        in_specs=[pl.BlockSpec((blk, H), lambda i: (i, 0)),
                  pl.BlockSpec((blk, H), lambda i: (i, 0)),
                  pl.BlockSpec((H, H), lambda i: (0, 0)),
                  pl.BlockSpec((1, H), lambda i: (0, 0)),
                  pl.BlockSpec((blk, H), lambda i: (i, 0))],
        out_specs=[pl.BlockSpec((blk, H), lambda i: (i, 0)),
                   pl.BlockSpec((8, H), lambda i: (0, 0))],
        out_shape=[jax.ShapeDtypeStruct((N, H), jnp.float32),
                   jax.ShapeDtypeStruct((8, H), jnp.float32)],
    )(g, aggr, w, b.reshape(1, -1), res)


def _bnrelu_body(relu, h_ref, s_ref, gm_ref, bt_ref, o_ref):
    mu = s_ref[0:1, :] / N
    var = s_ref[1:2, :] / N - mu * mu
    rstd = lax.rsqrt(var + 1e-5)
    o = (h_ref[...] - mu) * rstd * gm_ref[...] + bt_ref[...]
    if relu:
        o = jnp.maximum(o, 0.0)
    o_ref[...] = o


def _bnrelu(h, stats, gm, bt, relu, blk=1000):
    return pl.pallas_call(
        functools.partial(_bnrelu_body, relu),
        grid=(N // blk,),
        in_specs=[pl.BlockSpec((blk, H), lambda i: (i, 0)),
                  pl.BlockSpec((8, H), lambda i: (0, 0)),
                  pl.BlockSpec((1, H), lambda i: (0, 0)),
                  pl.BlockSpec((1, H), lambda i: (0, 0))],
        out_specs=pl.BlockSpec((blk, H), lambda i: (i, 0)),
        out_shape=jax.ShapeDtypeStruct((N, H), jnp.float32),
    )(h, stats, gm.reshape(1, -1), bt.reshape(1, -1))


def _pool_body(msel_ref, h_ref, wp_ref, bp_ref, o_ref, acc, cnt):
    @pl.when(pl.program_id(0) == 0)
    def _():
        acc[...] = jnp.zeros_like(acc)
        cnt[...] = jnp.zeros_like(cnt)

    msel_t = msel_ref[...]
    acc[...] = acc[...] + lax.dot_general(
        msel_t, h_ref[...], (((0,), (0,)), ((), ())),
        preferred_element_type=jnp.float32)
    cnt[...] = cnt[...] + jnp.broadcast_to(
        jnp.sum(msel_t, axis=0)[:, None], cnt.shape)

    @pl.when(pl.program_id(0) == pl.num_programs(0) - 1)
    def _():
        hg = acc[...] / jnp.maximum(cnt[...][:, 0:1], 1.0)
        o_ref[...] = (jnp.dot(hg, wp_ref[...],
                              preferred_element_type=jnp.float32) + bp_ref[...])


def _pool_proj(msel, hn, wp, bp, blk=2000):
    t_out = wp.shape[1]
    return pl.pallas_call(
        _pool_body,
        grid=(N // blk,),
        in_specs=[pl.BlockSpec((blk, G), lambda i: (i, 0)),
                  pl.BlockSpec((blk, H), lambda i: (i, 0)),
                  pl.BlockSpec((H, t_out), lambda i: (0, 0)),
                  pl.BlockSpec((1, t_out), lambda i: (0, 0))],
        out_specs=pl.BlockSpec((G, t_out), lambda i: (0, 0)),
        out_shape=jax.ShapeDtypeStruct((G, t_out), jnp.float32),
        scratch_shapes=[pltpu.VMEM((G, H), jnp.float32),
                        pltpu.VMEM((G, 128), jnp.float32)],
    )(msel, hn, wp, bp.reshape(1, -1))


def kernel(x, edge_index, edge_attr, batch, W0, b0, We, be, gcn_W, gcn_b, t,
           gamma, beta, Wp, bp):
    L = gcn_W.shape[0]
    src = edge_index[0]
    dst = edge_index[1]

    perm = jnp.argsort(dst)
    dst_s = dst[perm].astype(jnp.int32)
    src_s = src[perm].astype(jnp.int32)
    ea_s = edge_attr[perm]
    row_off = jnp.searchsorted(dst_s, jnp.arange(N + 1, dtype=jnp.int32)
                               ).astype(jnp.int32)
    roff_pad = jnp.concatenate(
        [row_off, jnp.full((RB,), E, jnp.int32)])
    starts = jnp.array([16 * g for g in _GSTART], dtype=jnp.int32)
    eo = row_off[starts]
    gidx = jnp.array(_GSTART, dtype=jnp.int32)
    eoff = jnp.zeros((80,), jnp.int32)
    eoff = eoff.at[0:66:2].set(eo).at[1:66:2].set(gidx)

    h0 = _mm(x, W0, b0, blk=1000)
    ee = _mm(ea_s, We, be, blk=2000)

    res = jnp.zeros((N, H), jnp.float32)
    g = h0
    h = None
    stats = None
    for l in range(L):
        tvec = jnp.full((16,), t[l], jnp.float32)
        aggr = _sc_agg(g, ee, src_s, dst_s, roff_pad, eoff, tvec)
        h, stats = _layer(g, aggr, gcn_W[l], gcn_b[l], res)
        res = h
        if l < L - 1:
            g = _bnrelu(h, stats, gamma[l], beta[l], relu=True)
    hn = _bnrelu(h, stats, gamma[L - 1], beta[L - 1], relu=False)

    msel_t = (batch[:, None] == jnp.arange(G, dtype=batch.dtype)[None, :]
              ).astype(jnp.float32)
    return _pool_proj(msel_t, hn, Wp, bp)


# speedup vs baseline: 2.3773x; 1.6019x over previous
import functools

import jax
import jax.numpy as jnp
from jax import lax
from jax.experimental import pallas as pl
from jax.experimental.pallas import tpu as pltpu
from jax.experimental.pallas import tpu_sc as plsc

N = 10000
E = 320000
H = 256
G = 64
NWORK = 32
NGRP = 625
EB = 128
RB = 336
NEG = -1e30

_GSTART = [(w * NGRP) // NWORK for w in range(NWORK + 1)]


def _sc_agg_body(g_hbm, ee_hbm, src_hbm, dst_hbm, roff_hbm, eoff_hbm,
                 tvec_hbm, out_hbm,
                 eoff_v, tvec_v, roff_v, src_v, dst_v, rows_v, ee_v,
                 rmax_v, den_v, num_v, out_v, sem):
    c = lax.axis_index("c")
    s = lax.axis_index("s")
    wid = s * 2 + c

    pltpu.sync_copy(eoff_hbm, eoff_v)
    pltpu.sync_copy(tvec_hbm, tvec_v)
    tv = tvec_v[...]

    ev = eoff_v[pl.ds(2 * wid, 16)]
    e0 = ev[0]
    g0 = ev[1]
    e1 = ev[2]
    g1 = ev[3]
    n0 = g0 * 16

    pltpu.sync_copy(roff_hbm.at[pl.ds(n0, RB)], roff_v)

    zv = jnp.zeros((16,), jnp.float32)
    negv = jnp.full((16,), NEG, jnp.float32)

    def _zero_out(i, _):
        for cg in range(16):
            out_v[i, pl.ds(cg * 16, 16)] = zv
        return 0

    lax.fori_loop(0, 16, _zero_out, 0)

    def _zfill(k, _):
        pltpu.sync_copy(out_v, out_hbm.at[pl.ds((g0 + k) * 16, 16)])
        return 0

    lax.fori_loop(0, g1 - g0, _zfill, 0)

    def _reset_acc():
        for g in range(16):
            rmax_v[g] = negv
            den_v[g] = zv
            num_v[g] = zv

    e_al = (e0 // EB) * EB
    nblk = (e1 - e_al + EB - 1) // EB

    def _block(k, car):
        cur, curg = car
        base = e_al + k * EB
        pltpu.sync_copy(src_hbm.at[pl.ds(base, EB)], src_v)
        pltpu.sync_copy(dst_hbm.at[pl.ds(base, EB)], dst_v.at[pl.ds(0, EB)])
        pltpu.async_copy(g_hbm.at[src_v], rows_v, sem).wait()
        pltpu.sync_copy(ee_hbm.at[pl.ds(base, EB)], ee_v)
        j_lo = jnp.maximum(e0 - base, 0)
        j_hi = jnp.minimum(e1 - base, EB)
        last = dst_v[pl.ds(j_hi - 1, 16)][0]

        def _node(nd, cg):
            rv = roff_v[pl.ds(nd - n0, 16)]
            es = rv[0]
            ee2 = rv[1]
            js = jnp.maximum(es - base, j_lo)
            je = jnp.minimum(ee2 - base, j_hi)
            ng = nd // 16

            @pl.when(ng != cg)
            def _():
                pltpu.sync_copy(out_v, out_hbm.at[pl.ds(cg * 16, 16)])
                lax.fori_loop(0, 16, _zero_out, 0)

            @pl.when(es >= base + j_lo)
            def _():
                _reset_acc()

            acc0 = (tuple(rmax_v[g] for g in range(16))
                    + tuple(den_v[g] for g in range(16))
                    + tuple(num_v[g] for g in range(16)))

            def _edge(j, acc):
                rmax = list(acc[0:16])
                den = list(acc[16:32])
                num = list(acc[32:48])
                for g in range(16):
                    hrow = rows_v[j, pl.ds(g * 16, 16)]
                    erow = ee_v[j, pl.ds(g * 16, 16)]
                    m = jnp.maximum(hrow + erow, 0.0) + 1e-7
                    logit = m * tv
                    nm = jnp.maximum(rmax[g], logit)
                    a = jnp.exp(logit - nm)
                    sc = jnp.exp(rmax[g] - nm)
                    den[g] = den[g] * sc + a
                    num[g] = num[g] * sc + a * m
                    rmax[g] = nm
                return tuple(rmax) + tuple(den) + tuple(num)

            acc = lax.fori_loop(js, je, _edge, acc0)
            for g in range(16):
                rmax_v[g] = acc[g]
                den_v[g] = acc[16 + g]
                num_v[g] = acc[32 + g]

            @pl.when(jnp.logical_and(ee2 <= base + j_hi, ee2 > es))
            def _():
                r = nd - ng * 16
                for g in range(16):
                    out_v[r, pl.ds(g * 16, 16)] = num_v[g] / den_v[g]

            return ng

        curg = lax.fori_loop(cur, last + 1, _node, curg)
        rl = roff_v[pl.ds(last - n0, 16)]
        cur = lax.select(rl[1] <= base + EB, last + 1, last)
        return cur, curg

    cur, curg = lax.fori_loop(0, nblk, _block, (n0, g0))
    pltpu.sync_copy(out_v, out_hbm.at[pl.ds(curg * 16, 16)])


def _sc_agg(g, ee, src_s, dst_s, roff_pad, eoff, tvec):
    mesh = plsc.VectorSubcoreMesh(core_axis_name="c", subcore_axis_name="s",
                                  num_cores=2, num_subcores=16)
    return pl.kernel(
        _sc_agg_body,
        out_type=jax.ShapeDtypeStruct((N, H), jnp.float32),
        mesh=mesh,
        scratch_types=[
            pltpu.VMEM((80,), jnp.int32),
            pltpu.VMEM((16,), jnp.float32),
            pltpu.VMEM((RB,), jnp.int32),
            pltpu.VMEM((EB,), jnp.int32),
            pltpu.VMEM((EB + 16,), jnp.int32),
            pltpu.VMEM((EB, H), jnp.float32),
            pltpu.VMEM((EB, H), jnp.float32),
            pltpu.VMEM((16, 16), jnp.float32),
            pltpu.VMEM((16, 16), jnp.float32),
            pltpu.VMEM((16, 16), jnp.float32),
            pltpu.VMEM((16, H), jnp.float32),
            pltpu.SemaphoreType.DMA,
        ],
    )(g, ee, src_s, dst_s, roff_pad, eoff, tvec)


def _mm_body(x_ref, w_ref, b_ref, o_ref):
    o_ref[...] = (jnp.dot(x_ref[...], w_ref[...],
                          preferred_element_type=jnp.float32) + b_ref[...])


def _mm(x, w, b, blk):
    m, k = x.shape
    n = w.shape[1]
    return pl.pallas_call(
        _mm_body,
        grid=(m // blk,),
        in_specs=[pl.BlockSpec((blk, k), lambda i: (i, 0)),
                  pl.BlockSpec((k, n), lambda i: (0, 0)),
                  pl.BlockSpec((1, n), lambda i: (0, 0))],
        out_specs=pl.BlockSpec((blk, n), lambda i: (i, 0)),
        out_shape=jax.ShapeDtypeStruct((m, n), jnp.float32),
    )(x, w, b.reshape(1, -1))


def _layer_body(g_ref, a_ref, w_ref, b_ref, r_ref, h_ref, s_ref):
    h = (jnp.dot(g_ref[...] + a_ref[...], w_ref[...],
                 preferred_element_type=jnp.float32) + b_ref[...] + r_ref[...])
    h_ref[...] = h
    cs = jnp.sum(h, axis=0, keepdims=True)
    cq = jnp.sum(h * h, axis=0, keepdims=True)
    st = jnp.concatenate([cs, cq, jnp.zeros((6, h.shape[1]), jnp.float32)], 0)

    @pl.when(pl.program_id(0) == 0)
    def _():
        s_ref[...] = st

    @pl.when(pl.program_id(0) > 0)
    def _():
        s_ref[...] = s_ref[...] + st


def _layer(g, aggr, w, b, res, blk=1000):
    return pl.pallas_call(
        _layer_body,
        grid=(N // blk,),
        in_specs=[pl.BlockSpec((blk, H), lambda i: (i, 0)),
                  pl.BlockSpec((blk, H), lambda i: (i, 0)),
                  pl.BlockSpec((H, H), lambda i: (0, 0)),
                  pl.BlockSpec((1, H), lambda i: (0, 0)),
                  pl.BlockSpec((blk, H), lambda i: (i, 0))],
        out_specs=[pl.BlockSpec((blk, H), lambda i: (i, 0)),
                   pl.BlockSpec((8, H), lambda i: (0, 0))],
        out_shape=[jax.ShapeDtypeStruct((N, H), jnp.float32),
                   jax.ShapeDtypeStruct((8, H), jnp.float32)],
    )(g, aggr, w, b.reshape(1, -1), res)


def _bnrelu_body(relu, h_ref, s_ref, gm_ref, bt_ref, o_ref):
    mu = s_ref[0:1, :] / N
    var = s_ref[1:2, :] / N - mu * mu
    rstd = lax.rsqrt(var + 1e-5)
    o = (h_ref[...] - mu) * rstd * gm_ref[...] + bt_ref[...]
    if relu:
        o = jnp.maximum(o, 0.0)
    o_ref[...] = o


def _bnrelu(h, stats, gm, bt, relu, blk=1000):
    return pl.pallas_call(
        functools.partial(_bnrelu_body, relu),
        grid=(N // blk,),
        in_specs=[pl.BlockSpec((blk, H), lambda i: (i, 0)),
                  pl.BlockSpec((8, H), lambda i: (0, 0)),
                  pl.BlockSpec((1, H), lambda i: (0, 0)),
                  pl.BlockSpec((1, H), lambda i: (0, 0))],
        out_specs=pl.BlockSpec((blk, H), lambda i: (i, 0)),
        out_shape=jax.ShapeDtypeStruct((N, H), jnp.float32),
    )(h, stats, gm.reshape(1, -1), bt.reshape(1, -1))


def _pool_body(msel_ref, h_ref, wp_ref, bp_ref, o_ref, acc, cnt):
    @pl.when(pl.program_id(0) == 0)
    def _():
        acc[...] = jnp.zeros_like(acc)
        cnt[...] = jnp.zeros_like(cnt)

    msel_t = msel_ref[...]
    acc[...] = acc[...] + lax.dot_general(
        msel_t, h_ref[...], (((0,), (0,)), ((), ())),
        preferred_element_type=jnp.float32)
    cnt[...] = cnt[...] + jnp.broadcast_to(
        jnp.sum(msel_t, axis=0)[:, None], cnt.shape)

    @pl.when(pl.program_id(0) == pl.num_programs(0) - 1)
    def _():
        hg = acc[...] / jnp.maximum(cnt[...][:, 0:1], 1.0)
        o_ref[...] = (jnp.dot(hg, wp_ref[...],
                              preferred_element_type=jnp.float32) + bp_ref[...])


def _pool_proj(msel, hn, wp, bp, blk=2000):
    t_out = wp.shape[1]
    return pl.pallas_call(
        _pool_body,
        grid=(N // blk,),
        in_specs=[pl.BlockSpec((blk, G), lambda i: (i, 0)),
                  pl.BlockSpec((blk, H), lambda i: (i, 0)),
                  pl.BlockSpec((H, t_out), lambda i: (0, 0)),
                  pl.BlockSpec((1, t_out), lambda i: (0, 0))],
        out_specs=pl.BlockSpec((G, t_out), lambda i: (0, 0)),
        out_shape=jax.ShapeDtypeStruct((G, t_out), jnp.float32),
        scratch_shapes=[pltpu.VMEM((G, H), jnp.float32),
                        pltpu.VMEM((G, 128), jnp.float32)],
    )(msel, hn, wp, bp.reshape(1, -1))


def kernel(x, edge_index, edge_attr, batch, W0, b0, We, be, gcn_W, gcn_b, t,
           gamma, beta, Wp, bp):
    L = gcn_W.shape[0]
    src = edge_index[0]
    dst = edge_index[1]

    perm = jnp.argsort(dst)
    dst_s = dst[perm].astype(jnp.int32)
    src_s = src[perm].astype(jnp.int32)
    ea_s = edge_attr[perm]
    row_off = jnp.searchsorted(dst_s, jnp.arange(N + 1, dtype=jnp.int32)
                               ).astype(jnp.int32)
    roff_pad = jnp.concatenate(
        [row_off, jnp.full((RB,), E, jnp.int32)])
    starts = jnp.array([16 * g for g in _GSTART], dtype=jnp.int32)
    eo = row_off[starts]
    gidx = jnp.array(_GSTART, dtype=jnp.int32)
    eoff = jnp.zeros((80,), jnp.int32)
    eoff = eoff.at[0:66:2].set(eo).at[1:66:2].set(gidx)

    h0 = _mm(x, W0, b0, blk=1000)
    ee = _mm(ea_s, We, be, blk=2000)

    res = jnp.zeros((N, H), jnp.float32)
    g = h0
    h = None
    stats = None
    for l in range(L):
        tvec = jnp.full((16,), t[l], jnp.float32)
        aggr = _sc_agg(g, ee, src_s, dst_s, roff_pad, eoff, tvec)
        h, stats = _layer(g, aggr, gcn_W[l], gcn_b[l], res)
        res = h
        if l < L - 1:
            g = _bnrelu(h, stats, gamma[l], beta[l], relu=True)
    hn = _bnrelu(h, stats, gamma[L - 1], beta[L - 1], relu=False)

    msel_t = (batch[:, None] == jnp.arange(G, dtype=batch.dtype)[None, :]
              ).astype(jnp.float32)
    return _pool_proj(msel_t, hn, Wp, bp)


# speedup vs baseline: 3.3022x; 1.3890x over previous
import functools

import jax
import jax.numpy as jnp
from jax import lax
from jax.experimental import pallas as pl
from jax.experimental.pallas import tpu as pltpu
from jax.experimental.pallas import tpu_sc as plsc

N = 10000
E = 320000
H = 256
HH = 128
G = 64
NWORK = 32
NGRP = 625
EB = 128
RB = 336
NEG = -1e30

_GSTART = [(w * NGRP) // NWORK for w in range(NWORK + 1)]


def _sc_half(g_hbm, ee_hbm, out_hbm, src_hbm, dst_hbm,
             roff_v, src_v, dst_v, rows_v, ee_v, out_v, sem,
             tv, e0, e1, g0, g1, n0):
    zv = jnp.zeros((16,), jnp.float32)
    negv = jnp.full((16,), NEG, jnp.float32)

    def _zero_out(i, _):
        for cg in range(8):
            out_v[i, pl.ds(cg * 16, 16)] = zv
        return 0

    lax.fori_loop(0, 16, _zero_out, 0)

    def _zfill(k, _):
        pltpu.sync_copy(out_v, out_hbm.at[pl.ds((g0 + k) * 16, 16)])
        return 0

    lax.fori_loop(0, g1 - g0, _zfill, 0)

    e_al = (e0 // EB) * EB
    nblk = (e1 - e_al + EB - 1) // EB
    init_acc = (negv,) * 8 + (zv,) * 8 + (zv,) * 8

    def _block(k, car):
        base = e_al + k * EB
        pltpu.sync_copy(src_hbm.at[pl.ds(base, EB)], src_v)
        pltpu.sync_copy(dst_hbm.at[pl.ds(base, EB)], dst_v.at[pl.ds(0, EB)])
        pltpu.async_copy(g_hbm.at[src_v], rows_v, sem).wait()
        pltpu.sync_copy(ee_hbm.at[pl.ds(base, EB)], ee_v)
        j_lo = jnp.maximum(e0 - base, 0)
        j_hi = jnp.minimum(e1 - base, EB)
        last = dst_v[pl.ds(j_hi - 1, 16)][0]

        def _node(nd, car2):
            cg = car2[0]
            acc_in = car2[1:]
            rv = roff_v[pl.ds(nd - n0, 16)]
            es = rv[0]
            ee2 = rv[1]
            js = jnp.maximum(es - base, j_lo)
            je = jnp.minimum(ee2 - base, j_hi)
            ng = nd // 16

            @pl.when(ng != cg)
            def _():
                pltpu.sync_copy(out_v, out_hbm.at[pl.ds(cg * 16, 16)])
                lax.fori_loop(0, 16, _zero_out, 0)

            fresh = es >= base + j_lo
            acc0 = tuple(
                jnp.where(fresh, f, a)
                for f, a in zip(init_acc, acc_in))

            def _edge(j, acc):
                rmax = list(acc[0:8])
                den = list(acc[8:16])
                num = list(acc[16:24])
                for g in range(8):
                    hrow = rows_v[j, pl.ds(g * 16, 16)]
                    erow = ee_v[j, pl.ds(g * 16, 16)]
                    m = jnp.maximum(hrow + erow, 0.0) + 1e-7
                    logit = m * tv
                    nm = jnp.maximum(rmax[g], logit)
                    a = jnp.exp(logit - nm)
                    sc = jnp.exp(rmax[g] - nm)
                    den[g] = den[g] * sc + a
                    num[g] = num[g] * sc + a * m
                    rmax[g] = nm
                return tuple(rmax) + tuple(den) + tuple(num)

            acc = lax.fori_loop(js, je, _edge, acc0)

            @pl.when(jnp.logical_and(ee2 <= base + j_hi, ee2 > es))
            def _():
                r = nd - ng * 16
                for g in range(8):
                    out_v[r, pl.ds(g * 16, 16)] = acc[8 + g + 8] / acc[8 + g]

            return (ng,) + acc

        car2 = lax.fori_loop(car[0], last + 1, _node,
                             (car[1],) + tuple(car[2:]))
        curg = car2[0]
        acc = car2[1:]
        rl = roff_v[pl.ds(last - n0, 16)]
        cur = lax.select(rl[1] <= base + EB, last + 1, last)
        return (cur, curg) + tuple(acc)

    car = lax.fori_loop(0, nblk, _block, (n0, g0) + init_acc)
    pltpu.sync_copy(out_v, out_hbm.at[pl.ds(car[1] * 16, 16)])


def _sc_agg_body(glo_hbm, ghi_hbm, eelo_hbm, eehi_hbm, src_hbm, dst_hbm,
                 roff_hbm, eoff_hbm, tvec_hbm, outlo_hbm, outhi_hbm,
                 eoff_v, tvec_v, roff_v, src_v, dst_v, rows_v, ee_v,
                 out_v, sem):
    c = lax.axis_index("c")
    s = lax.axis_index("s")
    wid = s * 2 + c

    pltpu.sync_copy(eoff_hbm, eoff_v)
    pltpu.sync_copy(tvec_hbm, tvec_v)
    tv = tvec_v[...]

    ev = eoff_v[pl.ds(2 * wid, 16)]
    e0 = ev[0]
    g0 = ev[1]
    e1 = ev[2]
    g1 = ev[3]
    n0 = g0 * 16

    pltpu.sync_copy(roff_hbm.at[pl.ds(n0, RB)], roff_v)

    _sc_half(glo_hbm, eelo_hbm, outlo_hbm, src_hbm, dst_hbm,
             roff_v, src_v, dst_v, rows_v, ee_v, out_v, sem,
             tv, e0, e1, g0, g1, n0)
    _sc_half(ghi_hbm, eehi_hbm, outhi_hbm, src_hbm, dst_hbm,
             roff_v, src_v, dst_v, rows_v, ee_v, out_v, sem,
             tv, e0, e1, g0, g1, n0)


def _sc_agg(glo, ghi, eelo, eehi, src_s, dst_s, roff_pad, eoff, tvec):
    mesh = plsc.VectorSubcoreMesh(core_axis_name="c", subcore_axis_name="s",
                                  num_cores=2, num_subcores=16)
    return pl.kernel(
        _sc_agg_body,
        out_type=[jax.ShapeDtypeStruct((N, HH), jnp.float32),
                  jax.ShapeDtypeStruct((N, HH), jnp.float32)],
        mesh=mesh,
        scratch_types=[
            pltpu.VMEM((80,), jnp.int32),
            pltpu.VMEM((16,), jnp.float32),
            pltpu.VMEM((RB,), jnp.int32),
            pltpu.VMEM((EB,), jnp.int32),
            pltpu.VMEM((EB + 16,), jnp.int32),
            pltpu.VMEM((EB, HH), jnp.float32),
            pltpu.VMEM((EB, HH), jnp.float32),
            pltpu.VMEM((16, HH), jnp.float32),
            pltpu.SemaphoreType.DMA,
        ],
    )(glo, ghi, eelo, eehi, src_s, dst_s, roff_pad, eoff, tvec)


def _mm_body(x_ref, w_ref, b_ref, o_ref):
    o_ref[...] = (jnp.dot(x_ref[...], w_ref[...],
                          preferred_element_type=jnp.float32) + b_ref[...])


def _mm(x, w, b, blk):
    m, k = x.shape
    n = w.shape[1]
    return pl.pallas_call(
        _mm_body,
        grid=(m // blk,),
        in_specs=[pl.BlockSpec((blk, k), lambda i: (i, 0)),
                  pl.BlockSpec((k, n), lambda i: (0, 0)),
                  pl.BlockSpec((1, n), lambda i: (0, 0))],
        out_specs=pl.BlockSpec((blk, n), lambda i: (i, 0)),
        out_shape=jax.ShapeDtypeStruct((m, n), jnp.float32),
    )(x, w, b.reshape(1, -1))


def _layer_body(glo_ref, ghi_ref, alo_ref, ahi_ref, w_ref, b_ref, r_ref,
                h_ref, s_ref):
    hin = jnp.concatenate([glo_ref[...] + alo_ref[...],
                           ghi_ref[...] + ahi_ref[...]], axis=1)
    h = (jnp.dot(hin, w_ref[...], preferred_element_type=jnp.float32)
         + b_ref[...] + r_ref[...])
    h_ref[...] = h
    cs = jnp.sum(h, axis=0, keepdims=True)
    cq = jnp.sum(h * h, axis=0, keepdims=True)
    st = jnp.concatenate([cs, cq, jnp.zeros((6, h.shape[1]), jnp.float32)], 0)

    @pl.when(pl.program_id(0) == 0)
    def _():
        s_ref[...] = st

    @pl.when(pl.program_id(0) > 0)
    def _():
        s_ref[...] = s_ref[...] + st


def _layer(glo, ghi, alo, ahi, w, b, res, blk=1000):
    return pl.pallas_call(
        _layer_body,
        grid=(N // blk,),
        in_specs=[pl.BlockSpec((blk, HH), lambda i: (i, 0)),
                  pl.BlockSpec((blk, HH), lambda i: (i, 0)),
                  pl.BlockSpec((blk, HH), lambda i: (i, 0)),
                  pl.BlockSpec((blk, HH), lambda i: (i, 0)),
                  pl.BlockSpec((H, H), lambda i: (0, 0)),
                  pl.BlockSpec((1, H), lambda i: (0, 0)),
                  pl.BlockSpec((blk, H), lambda i: (i, 0))],
        out_specs=[pl.BlockSpec((blk, H), lambda i: (i, 0)),
                   pl.BlockSpec((8, H), lambda i: (0, 0))],
        out_shape=[jax.ShapeDtypeStruct((N, H), jnp.float32),
                   jax.ShapeDtypeStruct((8, H), jnp.float32)],
    )(glo, ghi, alo, ahi, w, b.reshape(1, -1), res)


def _bnrelu_body(relu, h_ref, s_ref, gm_ref, bt_ref, olo_ref, ohi_ref):
    mu = s_ref[0:1, :] / N
    var = s_ref[1:2, :] / N - mu * mu
    rstd = lax.rsqrt(var + 1e-5)
    o = (h_ref[...] - mu) * rstd * gm_ref[...] + bt_ref[...]
    if relu:
        o = jnp.maximum(o, 0.0)
    olo_ref[...] = o[:, :HH]
    ohi_ref[...] = o[:, HH:]


def _bnrelu(h, stats, gm, bt, relu, blk=1000):
    return pl.pallas_call(
        functools.partial(_bnrelu_body, relu),
        grid=(N // blk,),
        in_specs=[pl.BlockSpec((blk, H), lambda i: (i, 0)),
                  pl.BlockSpec((8, H), lambda i: (0, 0)),
                  pl.BlockSpec((1, H), lambda i: (0, 0)),
                  pl.BlockSpec((1, H), lambda i: (0, 0))],
        out_specs=[pl.BlockSpec((blk, HH), lambda i: (i, 0)),
                   pl.BlockSpec((blk, HH), lambda i: (i, 0))],
        out_shape=[jax.ShapeDtypeStruct((N, HH), jnp.float32),
                   jax.ShapeDtypeStruct((N, HH), jnp.float32)],
    )(h, stats, gm.reshape(1, -1), bt.reshape(1, -1))


def _pool_body(msel_ref, hlo_ref, hhi_ref, wp_ref, bp_ref, o_ref, acc, cnt):
    @pl.when(pl.program_id(0) == 0)
    def _():
        acc[...] = jnp.zeros_like(acc)
        cnt[...] = jnp.zeros_like(cnt)

    msel_t = msel_ref[...]
    h = jnp.concatenate([hlo_ref[...], hhi_ref[...]], axis=1)
    acc[...] = acc[...] + lax.dot_general(
        msel_t, h, (((0,), (0,)), ((), ())),
        preferred_element_type=jnp.float32)
    cnt[...] = cnt[...] + jnp.broadcast_to(
        jnp.sum(msel_t, axis=0)[:, None], cnt.shape)

    @pl.when(pl.program_id(0) == pl.num_programs(0) - 1)
    def _():
        hg = acc[...] / jnp.maximum(cnt[...][:, 0:1], 1.0)
        o_ref[...] = (jnp.dot(hg, wp_ref[...],
                              preferred_element_type=jnp.float32) + bp_ref[...])


def _pool_proj(msel, hlo, hhi, wp, bp, blk=2000):
    t_out = wp.shape[1]
    return pl.pallas_call(
        _pool_body,
        grid=(N // blk,),
        in_specs=[pl.BlockSpec((blk, G), lambda i: (i, 0)),
                  pl.BlockSpec((blk, HH), lambda i: (i, 0)),
                  pl.BlockSpec((blk, HH), lambda i: (i, 0)),
                  pl.BlockSpec((H, t_out), lambda i: (0, 0)),
                  pl.BlockSpec((1, t_out), lambda i: (0, 0))],
        out_specs=pl.BlockSpec((G, t_out), lambda i: (0, 0)),
        out_shape=jax.ShapeDtypeStruct((G, t_out), jnp.float32),
        scratch_shapes=[pltpu.VMEM((G, H), jnp.float32),
                        pltpu.VMEM((G, 128), jnp.float32)],
    )(msel, hlo, hhi, wp, bp.reshape(1, -1))


def kernel(x, edge_index, edge_attr, batch, W0, b0, We, be, gcn_W, gcn_b, t,
           gamma, beta, Wp, bp):
    L = gcn_W.shape[0]
    src = edge_index[0]
    dst = edge_index[1]

    perm = jnp.argsort(dst)
    dst_s = dst[perm].astype(jnp.int32)
    src_s = src[perm].astype(jnp.int32)
    ea_s = edge_attr[perm]
    row_off = jnp.searchsorted(dst_s, jnp.arange(N + 1, dtype=jnp.int32)
                               ).astype(jnp.int32)
    roff_pad = jnp.concatenate(
        [row_off, jnp.full((RB,), E, jnp.int32)])
    starts = jnp.array([16 * g for g in _GSTART], dtype=jnp.int32)
    eo = row_off[starts]
    gidx = jnp.array(_GSTART, dtype=jnp.int32)
    eoff = jnp.zeros((80,), jnp.int32)
    eoff = eoff.at[0:66:2].set(eo).at[1:66:2].set(gidx)

    glo = _mm(x, W0[:, :HH], b0[:HH], blk=1000)
    ghi = _mm(x, W0[:, HH:], b0[HH:], blk=1000)
    eelo = _mm(ea_s, We[:, :HH], be[:HH], blk=2000)
    eehi = _mm(ea_s, We[:, HH:], be[HH:], blk=2000)

    res = jnp.zeros((N, H), jnp.float32)
    h = None
    stats = None
    for l in range(L):
        tvec = jnp.full((16,), t[l], jnp.float32)
        alo, ahi = _sc_agg(glo, ghi, eelo, eehi, src_s, dst_s,
                           roff_pad, eoff, tvec)
        h, stats = _layer(glo, ghi, alo, ahi, gcn_W[l], gcn_b[l], res)
        res = h
        if l < L - 1:
            glo, ghi = _bnrelu(h, stats, gamma[l], beta[l], relu=True)
    hlo, hhi = _bnrelu(h, stats, gamma[L - 1], beta[L - 1], relu=False)

    msel_t = (batch[:, None] == jnp.arange(G, dtype=batch.dtype)[None, :]
              ).astype(jnp.float32)
    return _pool_proj(msel_t, hlo, hhi, Wp, bp)


# speedup vs baseline: 3.6111x; 1.0935x over previous
import functools

import jax
import jax.numpy as jnp
from jax import lax
from jax.experimental import pallas as pl
from jax.experimental.pallas import tpu as pltpu
from jax.experimental.pallas import tpu_sc as plsc

N = 10000
E = 320000
H = 256
HH = 128
G = 64
NWORK = 32
NGRP = 625
EB = 128
RB = 336
NEG = -1e30

_GSTART = [(w * NGRP) // NWORK for w in range(NWORK + 1)]


def _sc_half(g_hbm, ee_hbm, out_hbm, src_hbm, dst_hbm,
             roff_v, src_v, dst_v, rows_v, ee_v, out_v, sem,
             tv, e0, e1, g0, g1, n0):
    zv = jnp.zeros((16,), jnp.float32)
    negv = jnp.full((16,), NEG, jnp.float32)

    def _zero_out(i, _):
        for cg in range(8):
            out_v[i, pl.ds(cg * 16, 16)] = zv
        return 0

    lax.fori_loop(0, 16, _zero_out, 0)

    def _zfill(k, _):
        pltpu.sync_copy(out_v, out_hbm.at[pl.ds((g0 + k) * 16, 16)])
        return 0

    lax.fori_loop(0, g1 - g0, _zfill, 0)

    e_al = (e0 // EB) * EB
    nblk = (e1 - e_al + EB - 1) // EB
    init_acc = (negv,) * 8 + (zv,) * 8 + (zv,) * 8

    def _block(k, car):
        base = e_al + k * EB
        pltpu.sync_copy(src_hbm.at[pl.ds(base, EB)], src_v)
        cp_dst = pltpu.async_copy(dst_hbm.at[pl.ds(base, EB)],
                                  dst_v.at[pl.ds(0, EB)], sem)
        cp_rows = pltpu.async_copy(g_hbm.at[src_v], rows_v, sem)
        cp_ee = pltpu.async_copy(ee_hbm.at[pl.ds(base, EB)], ee_v, sem)
        cp_dst.wait()
        cp_rows.wait()
        cp_ee.wait()
        j_lo = jnp.maximum(e0 - base, 0)
        j_hi = jnp.minimum(e1 - base, EB)
        last = dst_v[pl.ds(j_hi - 1, 16)][0]

        def _node(nd, car2):
            cg = car2[0]
            acc_in = car2[1:]
            rv = roff_v[pl.ds(nd - n0, 16)]
            es = rv[0]
            ee2 = rv[1]
            js = jnp.maximum(es - base, j_lo)
            je = jnp.minimum(ee2 - base, j_hi)
            ng = nd // 16

            @pl.when(ng != cg)
            def _():
                pltpu.sync_copy(out_v, out_hbm.at[pl.ds(cg * 16, 16)])
                lax.fori_loop(0, 16, _zero_out, 0)

            fresh = es >= base + j_lo
            acc0 = tuple(
                jnp.where(fresh, f, a)
                for f, a in zip(init_acc, acc_in))

            def _edge(j, acc):
                rmax = list(acc[0:8])
                den = list(acc[8:16])
                num = list(acc[16:24])
                for g in range(8):
                    hrow = rows_v[j, pl.ds(g * 16, 16)]
                    erow = ee_v[j, pl.ds(g * 16, 16)]
                    m = jnp.maximum(hrow + erow, 0.0) + 1e-7
                    logit = m * tv
                    nm = jnp.maximum(rmax[g], logit)
                    a = jnp.exp(logit - nm)
                    sc = jnp.exp(rmax[g] - nm)
                    den[g] = den[g] * sc + a
                    num[g] = num[g] * sc + a * m
                    rmax[g] = nm
                return tuple(rmax) + tuple(den) + tuple(num)

            acc = lax.fori_loop(js, je, _edge, acc0)

            @pl.when(jnp.logical_and(ee2 <= base + j_hi, ee2 > es))
            def _():
                r = nd - ng * 16
                for g in range(8):
                    out_v[r, pl.ds(g * 16, 16)] = acc[8 + g + 8] / acc[8 + g]

            return (ng,) + acc

        car2 = lax.fori_loop(car[0], last + 1, _node,
                             (car[1],) + tuple(car[2:]))
        curg = car2[0]
        acc = car2[1:]
        rl = roff_v[pl.ds(last - n0, 16)]
        cur = lax.select(rl[1] <= base + EB, last + 1, last)
        return (cur, curg) + tuple(acc)

    car = lax.fori_loop(0, nblk, _block, (n0, g0) + init_acc)
    pltpu.sync_copy(out_v, out_hbm.at[pl.ds(car[1] * 16, 16)])


def _sc_agg_body(glo_hbm, ghi_hbm, eelo_hbm, eehi_hbm, src_hbm, dst_hbm,
                 roff_hbm, eoff_hbm, tvec_hbm, outlo_hbm, outhi_hbm,
                 eoff_v, tvec_v, roff_v, src_v, dst_v, rows_v, ee_v,
                 out_v, sem):
    c = lax.axis_index("c")
    s = lax.axis_index("s")
    wid = s * 2 + c

    pltpu.sync_copy(eoff_hbm, eoff_v)
    pltpu.sync_copy(tvec_hbm, tvec_v)
    tv = tvec_v[...]

    ev = eoff_v[pl.ds(2 * wid, 16)]
    e0 = ev[0]
    g0 = ev[1]
    e1 = ev[2]
    g1 = ev[3]
    n0 = g0 * 16

    pltpu.sync_copy(roff_hbm.at[pl.ds(n0, RB)], roff_v)

    _sc_half(glo_hbm, eelo_hbm, outlo_hbm, src_hbm, dst_hbm,
             roff_v, src_v, dst_v, rows_v, ee_v, out_v, sem,
             tv, e0, e1, g0, g1, n0)
    _sc_half(ghi_hbm, eehi_hbm, outhi_hbm, src_hbm, dst_hbm,
             roff_v, src_v, dst_v, rows_v, ee_v, out_v, sem,
             tv, e0, e1, g0, g1, n0)


def _sc_agg(glo, ghi, eelo, eehi, src_s, dst_s, roff_pad, eoff, tvec):
    mesh = plsc.VectorSubcoreMesh(core_axis_name="c", subcore_axis_name="s",
                                  num_cores=2, num_subcores=16)
    return pl.kernel(
        _sc_agg_body,
        out_type=[jax.ShapeDtypeStruct((N, HH), jnp.float32),
                  jax.ShapeDtypeStruct((N, HH), jnp.float32)],
        mesh=mesh,
        scratch_types=[
            pltpu.VMEM((80,), jnp.int32),
            pltpu.VMEM((16,), jnp.float32),
            pltpu.VMEM((RB,), jnp.int32),
            pltpu.VMEM((EB,), jnp.int32),
            pltpu.VMEM((EB + 16,), jnp.int32),
            pltpu.VMEM((EB, HH), jnp.float32),
            pltpu.VMEM((EB, HH), jnp.float32),
            pltpu.VMEM((16, HH), jnp.float32),
            pltpu.SemaphoreType.DMA,
        ],
    )(glo, ghi, eelo, eehi, src_s, dst_s, roff_pad, eoff, tvec)


def _mm_body(x_ref, w_ref, b_ref, o_ref):
    o_ref[...] = (jnp.dot(x_ref[...], w_ref[...],
                          preferred_element_type=jnp.float32) + b_ref[...])


def _mm(x, w, b, blk):
    m, k = x.shape
    n = w.shape[1]
    return pl.pallas_call(
        _mm_body,
        grid=(m // blk,),
        in_specs=[pl.BlockSpec((blk, k), lambda i: (i, 0)),
                  pl.BlockSpec((k, n), lambda i: (0, 0)),
                  pl.BlockSpec((1, n), lambda i: (0, 0))],
        out_specs=pl.BlockSpec((blk, n), lambda i: (i, 0)),
        out_shape=jax.ShapeDtypeStruct((m, n), jnp.float32),
    )(x, w, b.reshape(1, -1))


def _layer_body(glo_ref, ghi_ref, alo_ref, ahi_ref, w_ref, b_ref, r_ref,
                h_ref, s_ref):
    hin = jnp.concatenate([glo_ref[...] + alo_ref[...],
                           ghi_ref[...] + ahi_ref[...]], axis=1)
    h = (jnp.dot(hin, w_ref[...], preferred_element_type=jnp.float32)
         + b_ref[...] + r_ref[...])
    h_ref[...] = h
    cs = jnp.sum(h, axis=0, keepdims=True)
    cq = jnp.sum(h * h, axis=0, keepdims=True)
    st = jnp.concatenate([cs, cq, jnp.zeros((6, h.shape[1]), jnp.float32)], 0)

    @pl.when(pl.program_id(0) == 0)
    def _():
        s_ref[...] = st

    @pl.when(pl.program_id(0) > 0)
    def _():
        s_ref[...] = s_ref[...] + st


def _layer(glo, ghi, alo, ahi, w, b, res, blk=1000):
    return pl.pallas_call(
        _layer_body,
        grid=(N // blk,),
        in_specs=[pl.BlockSpec((blk, HH), lambda i: (i, 0)),
                  pl.BlockSpec((blk, HH), lambda i: (i, 0)),
                  pl.BlockSpec((blk, HH), lambda i: (i, 0)),
                  pl.BlockSpec((blk, HH), lambda i: (i, 0)),
                  pl.BlockSpec((H, H), lambda i: (0, 0)),
                  pl.BlockSpec((1, H), lambda i: (0, 0)),
                  pl.BlockSpec((blk, H), lambda i: (i, 0))],
        out_specs=[pl.BlockSpec((blk, H), lambda i: (i, 0)),
                   pl.BlockSpec((8, H), lambda i: (0, 0))],
        out_shape=[jax.ShapeDtypeStruct((N, H), jnp.float32),
                   jax.ShapeDtypeStruct((8, H), jnp.float32)],
    )(glo, ghi, alo, ahi, w, b.reshape(1, -1), res)


def _bnrelu_body(relu, h_ref, s_ref, gm_ref, bt_ref, olo_ref, ohi_ref):
    mu = s_ref[0:1, :] / N
    var = s_ref[1:2, :] / N - mu * mu
    rstd = lax.rsqrt(var + 1e-5)
    o = (h_ref[...] - mu) * rstd * gm_ref[...] + bt_ref[...]
    if relu:
        o = jnp.maximum(o, 0.0)
    olo_ref[...] = o[:, :HH]
    ohi_ref[...] = o[:, HH:]


def _bnrelu(h, stats, gm, bt, relu, blk=1000):
    return pl.pallas_call(
        functools.partial(_bnrelu_body, relu),
        grid=(N // blk,),
        in_specs=[pl.BlockSpec((blk, H), lambda i: (i, 0)),
                  pl.BlockSpec((8, H), lambda i: (0, 0)),
                  pl.BlockSpec((1, H), lambda i: (0, 0)),
                  pl.BlockSpec((1, H), lambda i: (0, 0))],
        out_specs=[pl.BlockSpec((blk, HH), lambda i: (i, 0)),
                   pl.BlockSpec((blk, HH), lambda i: (i, 0))],
        out_shape=[jax.ShapeDtypeStruct((N, HH), jnp.float32),
                   jax.ShapeDtypeStruct((N, HH), jnp.float32)],
    )(h, stats, gm.reshape(1, -1), bt.reshape(1, -1))


def _pool_body(msel_ref, hlo_ref, hhi_ref, wp_ref, bp_ref, o_ref, acc, cnt):
    @pl.when(pl.program_id(0) == 0)
    def _():
        acc[...] = jnp.zeros_like(acc)
        cnt[...] = jnp.zeros_like(cnt)

    msel_t = msel_ref[...]
    h = jnp.concatenate([hlo_ref[...], hhi_ref[...]], axis=1)
    acc[...] = acc[...] + lax.dot_general(
        msel_t, h, (((0,), (0,)), ((), ())),
        preferred_element_type=jnp.float32)
    cnt[...] = cnt[...] + jnp.broadcast_to(
        jnp.sum(msel_t, axis=0)[:, None], cnt.shape)

    @pl.when(pl.program_id(0) == pl.num_programs(0) - 1)
    def _():
        hg = acc[...] / jnp.maximum(cnt[...][:, 0:1], 1.0)
        o_ref[...] = (jnp.dot(hg, wp_ref[...],
                              preferred_element_type=jnp.float32) + bp_ref[...])


def _pool_proj(msel, hlo, hhi, wp, bp, blk=2000):
    t_out = wp.shape[1]
    return pl.pallas_call(
        _pool_body,
        grid=(N // blk,),
        in_specs=[pl.BlockSpec((blk, G), lambda i: (i, 0)),
                  pl.BlockSpec((blk, HH), lambda i: (i, 0)),
                  pl.BlockSpec((blk, HH), lambda i: (i, 0)),
                  pl.BlockSpec((H, t_out), lambda i: (0, 0)),
                  pl.BlockSpec((1, t_out), lambda i: (0, 0))],
        out_specs=pl.BlockSpec((G, t_out), lambda i: (0, 0)),
        out_shape=jax.ShapeDtypeStruct((G, t_out), jnp.float32),
        scratch_shapes=[pltpu.VMEM((G, H), jnp.float32),
                        pltpu.VMEM((G, 128), jnp.float32)],
    )(msel, hlo, hhi, wp, bp.reshape(1, -1))


def kernel(x, edge_index, edge_attr, batch, W0, b0, We, be, gcn_W, gcn_b, t,
           gamma, beta, Wp, bp):
    L = gcn_W.shape[0]
    src = edge_index[0]
    dst = edge_index[1]

    perm = jnp.argsort(dst)
    dst_s = dst[perm].astype(jnp.int32)
    src_s = src[perm].astype(jnp.int32)
    ea_s = edge_attr[perm]
    row_off = jnp.searchsorted(dst_s, jnp.arange(N + 1, dtype=jnp.int32)
                               ).astype(jnp.int32)
    roff_pad = jnp.concatenate(
        [row_off, jnp.full((RB,), E, jnp.int32)])
    starts = jnp.array([16 * g for g in _GSTART], dtype=jnp.int32)
    eo = row_off[starts]
    gidx = jnp.array(_GSTART, dtype=jnp.int32)
    eoff = jnp.zeros((80,), jnp.int32)
    eoff = eoff.at[0:66:2].set(eo).at[1:66:2].set(gidx)

    glo = _mm(x, W0[:, :HH], b0[:HH], blk=1000)
    ghi = _mm(x, W0[:, HH:], b0[HH:], blk=1000)
    eelo = _mm(ea_s, We[:, :HH], be[:HH], blk=2000)
    eehi = _mm(ea_s, We[:, HH:], be[HH:], blk=2000)

    res = jnp.zeros((N, H), jnp.float32)
    h = None
    stats = None
    for l in range(L):
        tvec = jnp.full((16,), t[l], jnp.float32)
        alo, ahi = _sc_agg(glo, ghi, eelo, eehi, src_s, dst_s,
                           roff_pad, eoff, tvec)
        h, stats = _layer(glo, ghi, alo, ahi, gcn_W[l], gcn_b[l], res)
        res = h
        if l < L - 1:
            glo, ghi = _bnrelu(h, stats, gamma[l], beta[l], relu=True)
    hlo, hhi = _bnrelu(h, stats, gamma[L - 1], beta[L - 1], relu=False)

    msel_t = (batch[:, None] == jnp.arange(G, dtype=batch.dtype)[None, :]
              ).astype(jnp.float32)
    return _pool_proj(msel_t, hlo, hhi, Wp, bp)
